# Initial kernel scaffold; baseline (speedup 1.0000x reference)
import jax
import jax.numpy as jnp
from jax.experimental import pallas as pl


def kernel(spec, unit_norm_state):
    raise NotImplementedError("write your pallas kernel here")


# speedup vs baseline: 35.5083x; 35.5083x over previous
import jax
import jax.numpy as jnp
import numpy as np
from jax.experimental import pallas as pl
from jax.experimental.pallas import tpu as pltpu

_N_FEAT = 256
_ALPHA = 0.95
_T = 8000
_C = 256
_K = -(-_T // _C)


def _ema_kernel(x_ref, a_ref, decay_ref, s0_ref, y_ref, s_carry):
    k = pl.program_id(1)

    @pl.when(k == 0)
    def _():
        s_carry[:] = s0_ref[:]

    x = x_ref[0, 0]
    c = jnp.abs(x) * (1.0 - _ALPHA)
    limit = _T - k * _C
    col = jax.lax.broadcasted_iota(jnp.int32, (_N_FEAT, _C), 1)
    c = jnp.where(col < limit, c, 0.0)

    u = jnp.dot(c, a_ref[:], preferred_element_type=jnp.float32)
    s = u + s_carry[:] * decay_ref[:]
    y_ref[0, 0] = x * jax.lax.rsqrt(s)
    s_carry[:] = s[:, _C - 1:_C]


def _make(interpret=False):
    tj = np.arange(_C)
    powm = np.where(tj[None, :] >= tj[:, None],
                    _ALPHA ** (tj[None, :] - tj[:, None]), 0.0).astype(np.float32)
    decay = (_ALPHA ** (tj + 1.0)).astype(np.float32).reshape(1, _C)

    def kfn(spec, unit_norm_state):
        B = spec.shape[0]
        s0 = jnp.reshape(unit_norm_state.astype(jnp.float32), (_N_FEAT, 1))
        return pl.pallas_call(
            _ema_kernel,
            grid=(B, _K),
            in_specs=[
                pl.BlockSpec((1, 1, _N_FEAT, _C), lambda b, k: (b, 0, 0, k)),
                pl.BlockSpec((_C, _C), lambda b, k: (0, 0)),
                pl.BlockSpec((1, _C), lambda b, k: (0, 0)),
                pl.BlockSpec((_N_FEAT, 1), lambda b, k: (0, 0)),
            ],
            out_specs=pl.BlockSpec((1, 1, _N_FEAT, _C), lambda b, k: (b, 0, 0, k)),
            out_shape=jax.ShapeDtypeStruct((B, 1, _N_FEAT, _T), jnp.float32),
            scratch_shapes=[pltpu.VMEM((_N_FEAT, 1), jnp.float32)],
            compiler_params=pltpu.CompilerParams(
                dimension_semantics=("parallel", "arbitrary"),
            ),
            name="ema_unit_norm",
            interpret=interpret,
        )(spec, jnp.asarray(powm), jnp.asarray(decay), s0)

    return kfn


def kernel(spec, unit_norm_state):
    return _make()(spec, unit_norm_state)


# speedup vs baseline: 60.4049x; 1.7011x over previous
import jax
import jax.numpy as jnp
import numpy as np
from jax.experimental import pallas as pl
from jax.experimental.pallas import tpu as pltpu

_N_FEAT = 256
_ALPHA = 0.95
_T = 8000
_C = 256
_CB = 2048
_NSUB = _CB // _C
_K = -(-_T // _CB)


def _ema_kernel(x_ref, a_ref, decay_ref, s0_ref, y_ref, s_carry):
    k = pl.program_id(1)

    @pl.when(k == 0)
    def _():
        s_carry[:] = s0_ref[:]

    a = a_ref[:]
    decay = decay_ref[:]
    col = jax.lax.broadcasted_iota(jnp.int32, (_N_FEAT, _C), 1)
    for i in range(_NSUB):
        sl = slice(i * _C, (i + 1) * _C)
        x = x_ref[0, 0, :, sl]
        c = jnp.abs(x) * (1.0 - _ALPHA)
        limit = _T - k * _CB - i * _C
        c = jnp.where(col < limit, c, 0.0)
        u = jnp.dot(c, a, preferred_element_type=jnp.float32)
        s = u + s_carry[:] * decay
        y_ref[0, 0, :, sl] = x * jax.lax.rsqrt(s)
        s_carry[:] = s[:, _C - 1:_C]


def _make(interpret=False):
    tj = np.arange(_C)
    powm = np.where(tj[None, :] >= tj[:, None],
                    _ALPHA ** (tj[None, :] - tj[:, None]), 0.0).astype(np.float32)
    decay = (_ALPHA ** (tj + 1.0)).astype(np.float32).reshape(1, _C)

    def kfn(spec, unit_norm_state):
        B = spec.shape[0]
        s0 = jnp.reshape(unit_norm_state.astype(jnp.float32), (_N_FEAT, 1))
        return pl.pallas_call(
            _ema_kernel,
            grid=(B, _K),
            in_specs=[
                pl.BlockSpec((1, 1, _N_FEAT, _CB), lambda b, k: (b, 0, 0, k)),
                pl.BlockSpec((_C, _C), lambda b, k: (0, 0)),
                pl.BlockSpec((1, _C), lambda b, k: (0, 0)),
                pl.BlockSpec((_N_FEAT, 1), lambda b, k: (0, 0)),
            ],
            out_specs=pl.BlockSpec((1, 1, _N_FEAT, _CB), lambda b, k: (b, 0, 0, k)),
            out_shape=jax.ShapeDtypeStruct((B, 1, _N_FEAT, _T), jnp.float32),
            scratch_shapes=[pltpu.VMEM((_N_FEAT, 1), jnp.float32)],
            compiler_params=pltpu.CompilerParams(
                dimension_semantics=("parallel", "arbitrary"),
            ),
            name="ema_unit_norm",
            interpret=interpret,
        )(spec, jnp.asarray(powm), jnp.asarray(decay), s0)

    return kfn


def kernel(spec, unit_norm_state):
    return _make()(spec, unit_norm_state)


# speedup vs baseline: 70.2871x; 1.1636x over previous
import jax
import jax.numpy as jnp
import numpy as np
from jax.experimental import pallas as pl
from jax.experimental.pallas import tpu as pltpu

_N_FEAT = 256
_ALPHA = 0.95
_T = 8000
_C = 256
_NFULL = _T // _C
_CTAIL = _T - _NFULL * _C


def _ema_kernel(x_ref, a_ref, decay_ref, s0_ref, y_ref):
    a = a_ref[:]
    decay = decay_ref[:]
    s_in = s0_ref[:]
    for i in range(_NFULL + 1):
        lo = i * _C
        w = _C if i < _NFULL else _CTAIL
        x = x_ref[0, 0, :, lo:lo + w]
        c = jnp.abs(x) * (1.0 - _ALPHA)
        u = jnp.dot(c, a[:w, :w], preferred_element_type=jnp.float32)
        s = u + s_in * decay[:, :w]
        y_ref[0, 0, :, lo:lo + w] = x * jax.lax.rsqrt(s)
        s_in = s[:, w - 1:w]


def _make(interpret=False):
    tj = np.arange(_C)
    powm = np.where(tj[None, :] >= tj[:, None],
                    _ALPHA ** (tj[None, :] - tj[:, None]), 0.0).astype(np.float32)
    decay = (_ALPHA ** (tj + 1.0)).astype(np.float32).reshape(1, _C)

    def kfn(spec, unit_norm_state):
        B = spec.shape[0]
        s0 = jnp.reshape(unit_norm_state.astype(jnp.float32), (_N_FEAT, 1))
        return pl.pallas_call(
            _ema_kernel,
            grid=(B,),
            in_specs=[
                pl.BlockSpec((1, 1, _N_FEAT, _T), lambda b: (b, 0, 0, 0)),
                pl.BlockSpec((_C, _C), lambda b: (0, 0)),
                pl.BlockSpec((1, _C), lambda b: (0, 0)),
                pl.BlockSpec((_N_FEAT, 1), lambda b: (0, 0)),
            ],
            out_specs=pl.BlockSpec((1, 1, _N_FEAT, _T), lambda b: (b, 0, 0, 0)),
            out_shape=jax.ShapeDtypeStruct((B, 1, _N_FEAT, _T), jnp.float32),
            compiler_params=pltpu.CompilerParams(
                dimension_semantics=("parallel",),
                vmem_limit_bytes=48 * 1024 * 1024,
            ),
            name="ema_unit_norm",
            interpret=interpret,
        )(spec, jnp.asarray(powm), jnp.asarray(decay), s0)

    return kfn


def kernel(spec, unit_norm_state):
    return _make()(spec, unit_norm_state)


# speedup vs baseline: 119.0288x; 1.6935x over previous
import jax
import jax.numpy as jnp
import numpy as np
from jax.experimental import pallas as pl
from jax.experimental.pallas import tpu as pltpu

_N_FEAT = 256
_ALPHA = 0.95
_T = 8000
_C = 256
_NFULL = _T // _C
_CTAIL = _T - _NFULL * _C


def _ema_kernel(x_ref, a_ref, decay_ref, s0_ref, y_ref):
    a = a_ref[:]
    decay = decay_ref[:]
    s_row = s0_ref[:]
    for i in range(_NFULL + 1):
        lo = i * _C
        w = _C if i < _NFULL else _CTAIL
        x = x_ref[0, 0, :, lo:lo + w]
        xt = jnp.transpose(x)
        ct = jnp.abs(xt) * (1.0 - _ALPHA)
        ut = jnp.dot(a[:w, :w], ct, preferred_element_type=jnp.float32)
        st = ut + decay[:w] * s_row
        y_ref[0, 0, lo:lo + w, :] = xt * jax.lax.rsqrt(st)
        s_row = st[w - 1:w, :]


def _make(interpret=False):
    tj = np.arange(_C)
    powm = np.where(tj[:, None] >= tj[None, :],
                    _ALPHA ** (tj[:, None] - tj[None, :]), 0.0).astype(np.float32)
    decay = (_ALPHA ** (tj + 1.0)).astype(np.float32).reshape(_C, 1)

    def kfn(spec, unit_norm_state):
        B = spec.shape[0]
        s0 = jnp.reshape(unit_norm_state.astype(jnp.float32), (1, _N_FEAT))
        out = pl.pallas_call(
            _ema_kernel,
            grid=(B,),
            in_specs=[
                pl.BlockSpec((1, 1, _N_FEAT, _T), lambda b: (b, 0, 0, 0)),
                pl.BlockSpec((_C, _C), lambda b: (0, 0)),
                pl.BlockSpec((_C, 1), lambda b: (0, 0)),
                pl.BlockSpec((1, _N_FEAT), lambda b: (0, 0)),
            ],
            out_specs=pl.BlockSpec((1, 1, _T, _N_FEAT), lambda b: (b, 0, 0, 0)),
            out_shape=jax.ShapeDtypeStruct((B, 1, _T, _N_FEAT), jnp.float32),
            compiler_params=pltpu.CompilerParams(
                dimension_semantics=("parallel",),
                vmem_limit_bytes=48 * 1024 * 1024,
            ),
            name="ema_unit_norm",
            interpret=interpret,
        )(spec, jnp.asarray(powm), jnp.asarray(decay), s0)
        return jnp.swapaxes(out, 2, 3)

    return kfn


def kernel(spec, unit_norm_state):
    return _make()(spec, unit_norm_state)
